# Initial kernel scaffold; baseline (speedup 1.0000x reference)
import jax
import jax.numpy as jnp
from jax import lax
from jax.experimental import pallas as pl
from jax.experimental.pallas import tpu as pltpu

_N, _D, _E, _R, _M, _G = 2048, 1024, 64, 16, 8, 8
_TILE = 256
_NEG = jnp.float32(-1e30)
_BIG = jnp.int32(1 << 30)


def _moe_body(h_ref, wcat_ref, bias_ref, w1_ref, w2_ref,
              out_ref, eid_ref, gate_ref, gidx_ref):
    h = h_ref[...]
    s = jnp.dot(h, wcat_ref[...], preferred_element_type=jnp.float32)
    s = s + bias_ref[0:1, :]
    lane = lax.broadcasted_iota(jnp.int32, s.shape, 1)

    gs = jnp.where(lane < _G, s, _NEG)
    gmax = jnp.max(gs, axis=1, keepdims=True)
    gidx = jnp.min(jnp.where(gs == gmax, lane, _BIG), axis=1, keepdims=True)

    in_local = (lane >= _G) & (lane < _G + _G * _M)
    grp_of_lane = jnp.where(in_local, (lane - _G) // _M, _BIG)
    ls = jnp.where(grp_of_lane == gidx, s, _NEG)
    v1 = jnp.max(ls, axis=1, keepdims=True)
    l1 = jnp.min(jnp.where(ls == v1, lane, _BIG), axis=1, keepdims=True)
    ls2 = jnp.where(lane == l1, _NEG, ls)
    v2 = jnp.max(ls2, axis=1, keepdims=True)
    l2 = jnp.min(jnp.where(ls2 == v2, lane, _BIG), axis=1, keepdims=True)
    e1 = l1 - _G
    e2 = l2 - _G
    t = jnp.exp(v2 - v1)
    g2 = t / (1.0 + t)
    g1 = 1.0 - g2

    tmp = jnp.dot(h, w1_ref[...], preferred_element_type=jnp.float32)
    tmp = jnp.maximum(tmp, 0.0)
    elane = lax.broadcasted_iota(jnp.int32, tmp.shape, 1) // _R
    wexp = jnp.where(elane == e1, g1, 0.0) + jnp.where(elane == e2, g2, 0.0)
    z = tmp * wexp
    out_ref[...] = jnp.dot(z, w2_ref[...], preferred_element_type=jnp.float32)

    eid_ref[...] = jnp.concatenate([e1, e2], axis=1)
    gate_ref[...] = jnp.concatenate([g1, g2], axis=1)
    gidx_ref[...] = gidx


def kernel(h, k, Wg, bg, local_router, W1, W2):
    f32 = jnp.float32
    wcat = jnp.zeros((_D, 128), f32)
    wcat = wcat.at[:, :_G].set(Wg.T)
    wcat = wcat.at[:, _G:_G + _G * _M].set(
        local_router.transpose(1, 0, 2).reshape(_D, _G * _M))
    bias = jnp.zeros((8, 128), f32).at[0, :_G].set(bg)
    w1t = W1.transpose(1, 0, 2).reshape(_D, _E * _R)
    w2f = W2.reshape(_E * _R, _D)

    grid = _N // _TILE
    out, eid, gate, gidx = pl.pallas_call(
        _moe_body,
        grid=(grid,),
        in_specs=[
            pl.BlockSpec((_TILE, _D), lambda i: (i, 0)),
            pl.BlockSpec((_D, 128), lambda i: (0, 0)),
            pl.BlockSpec((8, 128), lambda i: (0, 0)),
            pl.BlockSpec((_D, _E * _R), lambda i: (0, 0)),
            pl.BlockSpec((_E * _R, _D), lambda i: (0, 0)),
        ],
        out_specs=[
            pl.BlockSpec((_TILE, _D), lambda i: (i, 0)),
            pl.BlockSpec((_TILE, 2), lambda i: (i, 0)),
            pl.BlockSpec((_TILE, 2), lambda i: (i, 0)),
            pl.BlockSpec((_TILE, 1), lambda i: (i, 0)),
        ],
        out_shape=[
            jax.ShapeDtypeStruct((_N, _D), f32),
            jax.ShapeDtypeStruct((_N, 2), jnp.int32),
            jax.ShapeDtypeStruct((_N, 2), f32),
            jax.ShapeDtypeStruct((_N, 1), jnp.int32),
        ],
    )(h, wcat, bias, w1t, w2f)

    gate = gate + (jnp.asarray(k, gate.dtype) - 2.0)
    return out, eid, gate, gidx[:, 0]


# speedup vs baseline: 16.7129x; 16.7129x over previous
import jax
import jax.numpy as jnp
from jax import lax
from jax.experimental import pallas as pl
from jax.experimental.pallas import tpu as pltpu

_N, _D, _E, _R, _M, _G = 2048, 1024, 64, 16, 8, 8
_TILE = 256
_NEG = -1e30
_BIG = 1 << 30


def _moe_body(h_ref, wcat_ref, bias_ref, w1_ref, w2_ref,
              out_ref, eid_ref, gate_ref, gidx_ref):
    h = h_ref[...]
    s = jnp.dot(h, wcat_ref[...], preferred_element_type=jnp.float32)
    s = s + bias_ref[0:1, :]
    lane = lax.broadcasted_iota(jnp.int32, s.shape, 1)

    gs = jnp.where(lane < _G, s, _NEG)
    gmax = jnp.max(gs, axis=1, keepdims=True)
    gidx = jnp.min(jnp.where(gs == gmax, lane, _BIG), axis=1, keepdims=True)

    in_local = (lane >= _G) & (lane < _G + _G * _M)
    grp_of_lane = jnp.where(in_local, (lane - _G) // _M, _BIG)
    ls = jnp.where(grp_of_lane == gidx, s, _NEG)
    v1 = jnp.max(ls, axis=1, keepdims=True)
    l1 = jnp.min(jnp.where(ls == v1, lane, _BIG), axis=1, keepdims=True)
    ls2 = jnp.where(lane == l1, _NEG, ls)
    v2 = jnp.max(ls2, axis=1, keepdims=True)
    l2 = jnp.min(jnp.where(ls2 == v2, lane, _BIG), axis=1, keepdims=True)
    e1 = l1 - _G
    e2 = l2 - _G
    t = jnp.exp(v2 - v1)
    g2 = t / (1.0 + t)
    g1 = 1.0 - g2

    tmp = jnp.dot(h, w1_ref[...], preferred_element_type=jnp.float32)
    tmp = jnp.maximum(tmp, 0.0)
    elane = lax.broadcasted_iota(jnp.int32, tmp.shape, 1) // _R
    wexp = jnp.where(elane == e1, g1, 0.0) + jnp.where(elane == e2, g2, 0.0)
    z = tmp * wexp
    out_ref[...] = jnp.dot(z, w2_ref[...], preferred_element_type=jnp.float32)

    eid_ref[...] = jnp.concatenate([e1, e2], axis=1)
    gate_ref[...] = jnp.concatenate([g1, g2], axis=1)
    gidx_ref[...] = gidx


def kernel(h, k, Wg, bg, local_router, W1, W2):
    f32 = jnp.float32
    wcat = jnp.zeros((_D, 128), f32)
    wcat = wcat.at[:, :_G].set(Wg.T)
    wcat = wcat.at[:, _G:_G + _G * _M].set(
        local_router.transpose(1, 0, 2).reshape(_D, _G * _M))
    bias = jnp.zeros((8, 128), f32).at[0, :_G].set(bg)
    w1t = W1.transpose(1, 0, 2).reshape(_D, _E * _R)
    w2f = W2.reshape(_E * _R, _D)

    grid = _N // _TILE
    out, eid, gate, gidx = pl.pallas_call(
        _moe_body,
        grid=(grid,),
        in_specs=[
            pl.BlockSpec((_TILE, _D), lambda i: (i, 0)),
            pl.BlockSpec((_D, 128), lambda i: (0, 0)),
            pl.BlockSpec((8, 128), lambda i: (0, 0)),
            pl.BlockSpec((_D, _E * _R), lambda i: (0, 0)),
            pl.BlockSpec((_E * _R, _D), lambda i: (0, 0)),
        ],
        out_specs=[
            pl.BlockSpec((_TILE, _D), lambda i: (i, 0)),
            pl.BlockSpec((_TILE, 2), lambda i: (i, 0)),
            pl.BlockSpec((_TILE, 2), lambda i: (i, 0)),
            pl.BlockSpec((_TILE, 1), lambda i: (i, 0)),
        ],
        out_shape=[
            jax.ShapeDtypeStruct((_N, _D), f32),
            jax.ShapeDtypeStruct((_N, 2), jnp.int32),
            jax.ShapeDtypeStruct((_N, 2), f32),
            jax.ShapeDtypeStruct((_N, 1), jnp.int32),
        ],
    )(h, wcat, bias, w1t, w2f)

    gate = gate + (jnp.asarray(k, gate.dtype) - 2.0)
    return out, eid, gate, gidx[:, 0]
